# speedup vs baseline: 2.9764x; 2.9764x over previous
import functools

import jax
import jax.numpy as jnp
from jax import lax
from jax.experimental import pallas as pl
from jax.experimental.pallas import tpu as pltpu
from jax.experimental.pallas import tpu_sc as plsc

EMBED = 128
NW = 32
CHUNK = 128
LANES = 16


def _body(nchunk, num_buckets, x_hbm, table_hbm, out_hbm, idx_v, rows_v, sem):
    wid = lax.axis_index("s") * 2 + lax.axis_index("c")
    per_w = nchunk * CHUNK
    base = wid * per_w

    pltpu.sync_copy(x_hbm.at[wid], idx_v)

    def row(j, _):
        def vec(i, _):
            v = idx_v[j, pl.ds(i * LANES, LANES)]
            r = lax.rem(v, num_buckets) + 1
            idx_v[j, pl.ds(i * LANES, LANES)] = jnp.where(v == 0, 0, r)
            return 0
        return lax.fori_loop(0, CHUNK // LANES, vec, 0)

    lax.fori_loop(0, nchunk, row, 0)

    def chunk(j, _):
        pltpu.async_copy(table_hbm.at[idx_v.at[j]], rows_v, sem).wait()
        pltpu.sync_copy(rows_v, out_hbm.at[pl.ds(base + j * CHUNK, CHUNK)])
        return 0

    lax.fori_loop(0, nchunk, chunk, 0)


@functools.partial(jax.jit, static_argnames=("nchunk", "num_buckets"))
def _emb(x, table, nchunk, num_buckets):
    total = NW * nchunk * CHUNK
    mesh = plsc.VectorSubcoreMesh(core_axis_name="c", subcore_axis_name="s")
    k = functools.partial(
        pl.kernel,
        mesh=mesh,
        out_type=jax.ShapeDtypeStruct((total, EMBED), jnp.float32),
        scratch_types=[
            pltpu.VMEM((nchunk, CHUNK), jnp.int32),
            pltpu.VMEM((CHUNK, EMBED), jnp.float32),
            pltpu.SemaphoreType.DMA,
        ],
    )(functools.partial(_body, nchunk, num_buckets))
    return k(x, table)


def kernel(x, table):
    num_buckets = table.shape[0] - 1
    b, s = x.shape
    total = b * s
    nchunk = total // (NW * CHUNK)
    xi = x.astype(jnp.int32).reshape(NW, nchunk, CHUNK)
    out = _emb(xi, table, nchunk, num_buckets)
    return out.reshape(b, s, EMBED)


# speedup vs baseline: 3.3732x; 1.1333x over previous
import functools

import jax
import jax.numpy as jnp
from jax import lax
from jax.experimental import pallas as pl
from jax.experimental.pallas import tpu as pltpu
from jax.experimental.pallas import tpu_sc as plsc

EMBED = 128
NW = 32
CHUNK = 128
LANES = 16
NBUF = 6
LOOKAHEAD = 3


def _body(nchunk, num_buckets, x_hbm, table_hbm, out_hbm, idx_v, rows_v,
          gsem, osem):
    wid = lax.axis_index("s") * 2 + lax.axis_index("c")
    per_w = nchunk * CHUNK
    base = wid * per_w

    pltpu.sync_copy(x_hbm.at[wid], idx_v)

    def row(j, _):
        def vec(i, _):
            v = idx_v[j, pl.ds(i * LANES, LANES)]
            r = lax.rem(v, num_buckets) + 1
            idx_v[j, pl.ds(i * LANES, LANES)] = jnp.where(v == 0, 0, r)
            return 0
        return lax.fori_loop(0, CHUNK // LANES, vec, 0)

    lax.fori_loop(0, nchunk, row, 0)

    def gather(j, b):
        return pltpu.make_async_copy(
            table_hbm.at[idx_v.at[j]], rows_v.at[b], gsem.at[b])

    def out_copy(j, b):
        return pltpu.make_async_copy(
            rows_v.at[b], out_hbm.at[pl.ds(base + j * CHUNK, CHUNK)],
            osem.at[b])

    for k in range(LOOKAHEAD):
        gather(k, k).start()

    def chunk(j, _):
        g = j + LOOKAHEAD
        bg = lax.rem(g, NBUF)

        @pl.when(g < nchunk)
        def _():
            @pl.when(g >= NBUF)
            def _():
                out_copy(g - NBUF, bg).wait()
            gather(g, bg).start()

        b = lax.rem(j, NBUF)
        gather(j, b).wait()
        out_copy(j, b).start()
        return 0

    lax.fori_loop(0, nchunk, chunk, 0)

    for k in range(NBUF):
        j = nchunk - NBUF + k
        out_copy(j, j % NBUF).wait()


@functools.partial(jax.jit, static_argnames=("nchunk", "num_buckets"))
def _emb(x, table, nchunk, num_buckets):
    total = NW * nchunk * CHUNK
    mesh = plsc.VectorSubcoreMesh(core_axis_name="c", subcore_axis_name="s")
    k = functools.partial(
        pl.kernel,
        mesh=mesh,
        out_type=jax.ShapeDtypeStruct((total, EMBED), jnp.float32),
        scratch_types=[
            pltpu.VMEM((nchunk, CHUNK), jnp.int32),
            pltpu.VMEM((NBUF, CHUNK, EMBED), jnp.float32),
            pltpu.SemaphoreType.DMA((NBUF,)),
            pltpu.SemaphoreType.DMA((NBUF,)),
        ],
    )(functools.partial(_body, nchunk, num_buckets))
    return k(x, table)


def kernel(x, table):
    num_buckets = table.shape[0] - 1
    b, s = x.shape
    total = b * s
    nchunk = total // (NW * CHUNK)
    xi = x.astype(jnp.int32).reshape(NW, nchunk, CHUNK)
    out = _emb(xi, table, nchunk, num_buckets)
    return out.reshape(b, s, EMBED)


# speedup vs baseline: 5.3902x; 1.5980x over previous
import functools

import jax
import jax.numpy as jnp
from jax import lax
from jax.experimental import pallas as pl
from jax.experimental.pallas import tpu as pltpu
from jax.experimental.pallas import tpu_sc as plsc

EMBED = 128
NW = 32
LANES = 16
XPAD = 128
NBUF = 6
LOOKAHEAD = 3


def _body(nrow, rowlen, num_buckets, x_hbm, table_hbm, out_hbm, idx_v,
          rows_v, gsem, osem):
    wid = lax.axis_index("s") * 2 + lax.axis_index("c")
    row0 = wid * nrow

    pltpu.sync_copy(x_hbm.at[wid], idx_v)

    def trans(j, _):
        def vec(i, _):
            v = idx_v[j, pl.ds(i * LANES, LANES)]
            r = lax.rem(v, num_buckets) + 1
            idx_v[j, pl.ds(i * LANES, LANES)] = jnp.where(v == 0, 0, r)
            return 0
        return lax.fori_loop(0, XPAD // LANES, vec, 0)

    lax.fori_loop(0, nrow, trans, 0)

    def gather(r, b):
        return pltpu.make_async_copy(
            table_hbm.at[idx_v.at[r, pl.ds(0, rowlen)]], rows_v.at[b],
            gsem.at[b])

    def out_copy(r, b):
        return pltpu.make_async_copy(
            rows_v.at[b], out_hbm.at[row0 + r], osem.at[b])

    for k in range(LOOKAHEAD):
        gather(k, k).start()

    def step(r, _):
        g = r + LOOKAHEAD
        bg = lax.rem(g, NBUF)

        @pl.when(g < nrow)
        def _():
            @pl.when(g >= NBUF)
            def _():
                out_copy(g - NBUF, bg).wait()
            gather(g, bg).start()

        b = lax.rem(r, NBUF)
        gather(r, b).wait()
        out_copy(r, b).start()
        return 0

    lax.fori_loop(0, nrow, step, 0)

    for k in range(NBUF):
        r = nrow - NBUF + k
        out_copy(r, r % NBUF).wait()


@functools.partial(jax.jit, static_argnames=("nrow", "rowlen", "num_buckets"))
def _emb(x, table, nrow, rowlen, num_buckets):
    mesh = plsc.VectorSubcoreMesh(core_axis_name="c", subcore_axis_name="s")
    k = functools.partial(
        pl.kernel,
        mesh=mesh,
        out_type=jax.ShapeDtypeStruct((NW * nrow, rowlen, EMBED),
                                      jnp.float32),
        scratch_types=[
            pltpu.VMEM((nrow, XPAD), jnp.int32),
            pltpu.VMEM((NBUF, rowlen, EMBED), jnp.float32),
            pltpu.SemaphoreType.DMA((NBUF,)),
            pltpu.SemaphoreType.DMA((NBUF,)),
        ],
    )(functools.partial(_body, nrow, rowlen, num_buckets))
    return k(x, table)


def kernel(x, table):
    num_buckets = table.shape[0] - 1
    b, s = x.shape
    nrow = b // NW
    xi = jnp.pad(x.astype(jnp.int32), ((0, 0), (0, XPAD - s)))
    xi = xi.reshape(NW, nrow, XPAD)
    return _emb(xi, table, nrow, s, num_buckets)


# speedup vs baseline: 5.3917x; 1.0003x over previous
import functools

import jax
import jax.numpy as jnp
from jax import lax
from jax.experimental import pallas as pl
from jax.experimental.pallas import tpu as pltpu
from jax.experimental.pallas import tpu_sc as plsc

EMBED = 128
NW = 32
LANES = 16
XPAD = 128
NBUF = 6
LOOKAHEAD = 3


def _body(nrow, rowlen, num_buckets, x_hbm, table_hbm, out_hbm, idx_v,
          rows_v, gsem, osem):
    wid = lax.axis_index("s") * 2 + lax.axis_index("c")
    row0 = wid * nrow

    pltpu.sync_copy(x_hbm.at[wid], idx_v)

    def trans(j, _):
        def vec(i, _):
            v = idx_v[j, pl.ds(i * LANES, LANES)]
            r = lax.rem(v, num_buckets) + 1
            idx_v[j, pl.ds(i * LANES, LANES)] = jnp.where(v == 0, 0, r)
            return 0
        return lax.fori_loop(0, XPAD // LANES, vec, 0)

    lax.fori_loop(0, nrow, trans, 0)

    def gather(r, b):
        return pltpu.make_async_copy(
            table_hbm.at[idx_v.at[r, pl.ds(0, rowlen)]], rows_v.at[b],
            gsem.at[b])

    def out_copy(r, b):
        return pltpu.make_async_copy(
            rows_v.at[b], out_hbm.at[row0 + r], osem.at[b])

    for k in range(LOOKAHEAD):
        gather(k, k).start()

    def step(r, _):
        g = r + LOOKAHEAD
        bg = lax.rem(g, NBUF)

        @pl.when(g < nrow)
        def _():
            @pl.when(g >= NBUF)
            def _():
                out_copy(g - NBUF, bg).wait()
            gather(g, bg).start()

        b = lax.rem(r, NBUF)
        gather(r, b).wait()
        out_copy(r, b).start()
        return 0

    lax.fori_loop(0, nrow, step, 0)

    for k in range(NBUF):
        r = nrow - NBUF + k
        out_copy(r, r % NBUF).wait()


@functools.partial(jax.jit, static_argnames=("nrow", "rowlen", "num_buckets"))
def _emb(x, table, nrow, rowlen, num_buckets):
    mesh = plsc.VectorSubcoreMesh(core_axis_name="c", subcore_axis_name="s")
    k = functools.partial(
        pl.kernel,
        mesh=mesh,
        out_type=jax.ShapeDtypeStruct((NW * nrow, rowlen, EMBED),
                                      jnp.float32),
        scratch_types=[
            pltpu.VMEM((nrow, XPAD), jnp.int32),
            pltpu.VMEM((NBUF, rowlen, EMBED), jnp.float32),
            pltpu.SemaphoreType.DMA((NBUF,)),
            pltpu.SemaphoreType.DMA((NBUF,)),
        ],
        compiler_params=pltpu.CompilerParams(use_tc_tiling_on_sc=True),
    )(functools.partial(_body, nrow, rowlen, num_buckets))
    return k(x, table)


def kernel(x, table):
    num_buckets = table.shape[0] - 1
    b, s = x.shape
    nrow = b // NW
    xi = jnp.pad(x.astype(jnp.int32), ((0, 0), (0, XPAD - s)))
    xi = xi.reshape(NW, nrow, XPAD)
    return _emb(xi, table, nrow, s, num_buckets)


# speedup vs baseline: 9.2133x; 1.7088x over previous
import functools

import jax
import jax.numpy as jnp
from jax import lax
from jax.experimental import pallas as pl
from jax.experimental.pallas import tpu as pltpu
from jax.experimental.pallas import tpu_sc as plsc

EMBED = 128
NW = 32
LANES = 16
NBUF = 6
LOOKAHEAD = 3


def _body(ns, blk, num_buckets, xt_hbm, table_hbm, out_hbm, idx_v, rows_v,
          gsem, osem):
    wid = lax.axis_index("s") * 2 + lax.axis_index("c")
    col0 = wid * blk

    pltpu.sync_copy(xt_hbm.at[wid], idx_v)

    def trans(s, _):
        def vec(i, _):
            v = idx_v[s, pl.ds(i * LANES, LANES)]
            r = lax.rem(v, num_buckets) + 1
            idx_v[s, pl.ds(i * LANES, LANES)] = jnp.where(v == 0, 0, r)
            return 0
        return lax.fori_loop(0, blk // LANES, vec, 0)

    lax.fori_loop(0, ns, trans, 0)

    def gather(s, b):
        return pltpu.make_async_copy(
            table_hbm.at[idx_v.at[s]], rows_v.at[b], gsem.at[b])

    def out_copy(s, b):
        return pltpu.make_async_copy(
            rows_v.at[b], out_hbm.at[s, pl.ds(col0, blk)], osem.at[b])

    for k in range(LOOKAHEAD):
        gather(k, k).start()

    def step(s, _):
        g = s + LOOKAHEAD
        bg = lax.rem(g, NBUF)

        @pl.when(g < ns)
        def _():
            @pl.when(g >= NBUF)
            def _():
                out_copy(g - NBUF, bg).wait()
            gather(g, bg).start()

        b = lax.rem(s, NBUF)
        gather(s, b).wait()
        out_copy(s, b).start()
        return 0

    lax.fori_loop(0, ns, step, 0)

    for k in range(NBUF):
        s = ns - NBUF + k
        out_copy(s, s % NBUF).wait()


@functools.partial(jax.jit, static_argnames=("ns", "blk", "num_buckets"))
def _emb(xt, table, ns, blk, num_buckets):
    mesh = plsc.VectorSubcoreMesh(core_axis_name="c", subcore_axis_name="s")
    k = functools.partial(
        pl.kernel,
        mesh=mesh,
        out_type=jax.ShapeDtypeStruct((ns, NW * blk, EMBED), jnp.float32),
        scratch_types=[
            pltpu.VMEM((ns, blk), jnp.int32),
            pltpu.VMEM((NBUF, blk, EMBED), jnp.float32),
            pltpu.SemaphoreType.DMA((NBUF,)),
            pltpu.SemaphoreType.DMA((NBUF,)),
        ],
    )(functools.partial(_body, ns, blk, num_buckets))
    return k(xt, table)


def kernel(x, table):
    num_buckets = table.shape[0] - 1
    b, s = x.shape
    blk = b // NW
    xt = x.astype(jnp.int32).reshape(NW, blk, s).swapaxes(1, 2)
    out = _emb(xt, table, s, blk, num_buckets)
    return out.transpose(1, 0, 2)
